# baseline (speedup 1.0000x reference)
import functools

import numpy as np
import jax
import jax.numpy as jnp
from jax import lax
from jax.experimental import pallas as pl
from jax.experimental.pallas import tpu as pltpu
from jax.experimental.pallas import tpu_sc as plsc

D_MODEL = 768
MAX_SEQ_LEN = 4096
BATCH = 4
SEQ_LEN = 4096

_NC, _NS, _L = 2, 16, 16
_NW = _NC * _NS
_P = SEQ_LEN // _NW
_K = 32
_NCH = _P // _K
_VECS = D_MODEL // _L
_STEPS = _NCH * BATCH


def _pe_table():
    pos = np.arange(MAX_SEQ_LEN)[:, None].astype(np.float32)
    div_term = np.exp(
        np.arange(0, D_MODEL, 2).astype(np.float32) * (-np.log(10000.0) / D_MODEL)
    )
    pe = np.zeros((MAX_SEQ_LEN, D_MODEL), dtype=np.float32)
    pe[:, 0::2] = np.sin(pos * div_term)
    pe[:, 1::2] = np.cos(pos * div_term)
    return jnp.asarray(pe)


_PE = _pe_table()


def _body(x_hbm, pe_hbm, table_hbm, out_hbm, idx_v, pe_v, row_v, gsem, ssem, psem):
    cid = lax.axis_index("c")
    sid = lax.axis_index("s")
    wid = sid * _NC + cid
    pos0 = wid * _P

    for b in range(BATCH):
        pltpu.sync_copy(x_hbm.at[b, pl.ds(pos0, _P)], idx_v.at[b])

    def gather_desc(b, ch, buf):
        return pltpu.make_async_copy(
            table_hbm.at[idx_v.at[b, pl.ds(ch * _K, _K)]], row_v.at[buf],
            gsem.at[buf],
        )

    def pe_desc(ch, buf):
        return pltpu.make_async_copy(
            pe_hbm.at[pl.ds(pos0 + ch * _K, _K)], pe_v.at[buf], psem.at[buf]
        )

    def store_desc(b, ch, buf):
        return pltpu.make_async_copy(
            row_v.at[buf], out_hbm.at[b, pl.ds(pos0 + ch * _K, _K)], ssem.at[buf]
        )

    pe_desc(0, 0).start()
    gather_desc(0, 0, 0).start()
    gather_desc(1, 0, 1).start()

    @pl.loop(0, _STEPS)
    def _step(t):
        ch = t // BATCH
        b = t % BATCH
        buf = t % 3

        @pl.when(t < _STEPS - 2)
        def _():
            nt = t + 2
            nbuf = nt % 3

            @pl.when(t >= 1)
            def _():
                store_desc(0, 0, nbuf).wait()

            gather_desc(nt % BATCH, nt // BATCH, nbuf).start()

        @pl.when(b == 0)
        def _():
            @pl.when(ch + 1 < _NCH)
            def _():
                pe_desc(ch + 1, (ch + 1) % 2).start()

            pe_desc(0, ch % 2).wait()

        gather_desc(b, ch, buf).wait()

        pbuf = ch % 2
        _H = _K // 2

        @plsc.parallel_loop(0, _H, unroll=4)
        def _add_lo(i):
            for j in range(_VECS):
                vec = pe_v[pbuf, i, pl.ds(j * _L, _L)]
                plsc.addupdate(row_v.at[buf, i, pl.ds(j * _L, _L)], vec)

        pltpu.make_async_copy(
            row_v.at[buf, pl.ds(0, _H)],
            out_hbm.at[b, pl.ds(pos0 + ch * _K, _H)],
            ssem.at[buf],
        ).start()

        @plsc.parallel_loop(_H, _K, unroll=4)
        def _add_hi(i):
            for j in range(_VECS):
                vec = pe_v[pbuf, i, pl.ds(j * _L, _L)]
                plsc.addupdate(row_v.at[buf, i, pl.ds(j * _L, _L)], vec)

        pltpu.make_async_copy(
            row_v.at[buf, pl.ds(_H, _H)],
            out_hbm.at[b, pl.ds(pos0 + ch * _K + _H, _H)],
            ssem.at[buf],
        ).start()

    store_desc(0, 0, 0).wait()
    store_desc(0, 0, 1).wait()
    store_desc(0, 0, 2).wait()


@functools.partial(
    pl.kernel,
    out_type=jax.ShapeDtypeStruct((BATCH, SEQ_LEN, D_MODEL), jnp.float32),
    mesh=plsc.VectorSubcoreMesh(
        core_axis_name="c", subcore_axis_name="s", num_cores=_NC, num_subcores=_NS
    ),
    scratch_types=[
        pltpu.VMEM((BATCH, _P), jnp.int32),
        pltpu.VMEM((2, _K, D_MODEL), jnp.float32),
        pltpu.VMEM((3, _K, D_MODEL), jnp.float32),
        pltpu.SemaphoreType.DMA((3,)),
        pltpu.SemaphoreType.DMA((3,)),
        pltpu.SemaphoreType.DMA((2,)),
    ],
)
def _embed_sc(x_hbm, pe_hbm, table_hbm, out_hbm, idx_v, pe_v, row_v, gsem, ssem, psem):
    _body(x_hbm, pe_hbm, table_hbm, out_hbm, idx_v, pe_v, row_v, gsem, ssem, psem)


@jax.jit
def kernel(x, table):
    return _embed_sc(x, _PE, table)


# speedup vs baseline: 1.4281x; 1.4281x over previous
import functools

import numpy as np
import jax
import jax.numpy as jnp
from jax import lax
from jax.experimental import pallas as pl
from jax.experimental.pallas import tpu as pltpu
from jax.experimental.pallas import tpu_sc as plsc

D_MODEL = 768
MAX_SEQ_LEN = 4096
BATCH = 4
SEQ_LEN = 4096

_NC, _NS, _L = 2, 16, 16
_NW = _NC * _NS
_P = SEQ_LEN // _NW
_K = 32
_NCH = _P // _K
_VECS = D_MODEL // _L
_STEPS = _NCH * BATCH


def _pe_table():
    pos = np.arange(MAX_SEQ_LEN)[:, None].astype(np.float32)
    div_term = np.exp(
        np.arange(0, D_MODEL, 2).astype(np.float32) * (-np.log(10000.0) / D_MODEL)
    )
    pe = np.zeros((MAX_SEQ_LEN, D_MODEL), dtype=np.float32)
    pe[:, 0::2] = np.sin(pos * div_term)
    pe[:, 1::2] = np.cos(pos * div_term)
    return jnp.asarray(pe)


_PE = _pe_table()


def _body(x_hbm, pe_hbm, table_hbm, out_hbm, idx_v, pe_v, row_v, gsem, ssem, psem):
    cid = lax.axis_index("c")
    sid = lax.axis_index("s")
    wid = sid * _NC + cid
    pos0 = wid * _P

    for b in range(BATCH):
        pltpu.sync_copy(x_hbm.at[b, pl.ds(pos0, _P)], idx_v.at[b])

    def gather_desc(b, ch, buf):
        return pltpu.make_async_copy(
            table_hbm.at[idx_v.at[b, pl.ds(ch * _K, _K)]], row_v.at[buf],
            gsem.at[buf],
        )

    def pe_desc(ch, buf):
        return pltpu.make_async_copy(
            pe_hbm.at[pl.ds(pos0 + ch * _K, _K)], pe_v.at[buf], psem.at[buf]
        )

    def store_desc(b, ch, buf):
        return pltpu.make_async_copy(
            row_v.at[buf], out_hbm.at[b, pl.ds(pos0 + ch * _K, _K)], ssem.at[buf]
        )

    pe_desc(0, 0).start()
    gather_desc(0, 0, 0).start()
    gather_desc(1, 0, 1).start()

    @pl.loop(0, _STEPS)
    def _step(t):
        ch = t // BATCH
        b = t % BATCH
        buf = t % 3

        @pl.when(b == 0)
        def _():
            @pl.when(ch + 1 < _NCH)
            def _():
                pe_desc(ch + 1, (ch + 1) % 2).start()

            pe_desc(0, ch % 2).wait()

        gather_desc(b, ch, buf).wait()

        pbuf = ch % 2

        @plsc.parallel_loop(0, _K, unroll=4)
        def _add(i):
            for j in range(_VECS):
                vec = pe_v[pbuf, i, pl.ds(j * _L, _L)]
                plsc.addupdate(row_v.at[buf, i, pl.ds(j * _L, _L)], vec)

        store_desc(b, ch, buf).start()

        @pl.when(t < _STEPS - 2)
        def _():
            nt = t + 2
            nbuf = nt % 3

            @pl.when(t >= 1)
            def _():
                store_desc(0, 0, nbuf).wait()

            gather_desc(nt % BATCH, nt // BATCH, nbuf).start()

    store_desc(0, 0, 0).wait()
    store_desc(0, 0, 1).wait()
    store_desc(0, 0, 2).wait()


@functools.partial(
    pl.kernel,
    out_type=jax.ShapeDtypeStruct((BATCH, SEQ_LEN, D_MODEL), jnp.float32),
    mesh=plsc.VectorSubcoreMesh(
        core_axis_name="c", subcore_axis_name="s", num_cores=_NC, num_subcores=_NS
    ),
    scratch_types=[
        pltpu.VMEM((BATCH, _P), jnp.int32),
        pltpu.VMEM((2, _K, D_MODEL), jnp.float32),
        pltpu.VMEM((3, _K, D_MODEL), jnp.float32),
        pltpu.SemaphoreType.DMA((3,)),
        pltpu.SemaphoreType.DMA((3,)),
        pltpu.SemaphoreType.DMA((2,)),
    ],
)
def _embed_sc(x_hbm, pe_hbm, table_hbm, out_hbm, idx_v, pe_v, row_v, gsem, ssem, psem):
    _body(x_hbm, pe_hbm, table_hbm, out_hbm, idx_v, pe_v, row_v, gsem, ssem, psem)


@jax.jit
def kernel(x, table):
    return _embed_sc(x, _PE, table)


# speedup vs baseline: 1.4394x; 1.0079x over previous
import functools

import numpy as np
import jax
import jax.numpy as jnp
from jax import lax
from jax.experimental import pallas as pl
from jax.experimental.pallas import tpu as pltpu
from jax.experimental.pallas import tpu_sc as plsc

D_MODEL = 768
MAX_SEQ_LEN = 4096
BATCH = 4
SEQ_LEN = 4096

_NC, _NS, _L = 2, 16, 16
_NW = _NC * _NS
_P = SEQ_LEN // _NW
_K = 32
_NCH = _P // _K
_VECS = D_MODEL // _L
_STEPS = _NCH * BATCH


def _pe_table():
    pos = np.arange(MAX_SEQ_LEN)[:, None].astype(np.float32)
    div_term = np.exp(
        np.arange(0, D_MODEL, 2).astype(np.float32) * (-np.log(10000.0) / D_MODEL)
    )
    pe = np.zeros((MAX_SEQ_LEN, D_MODEL), dtype=np.float32)
    pe[:, 0::2] = np.sin(pos * div_term)
    pe[:, 1::2] = np.cos(pos * div_term)
    return jnp.asarray(pe)


_PE = _pe_table()


def _body(x_hbm, pe_hbm, table_hbm, out_hbm, idx_v, pe_v, row_v, gsem, ssem, psem, isem):
    cid = lax.axis_index("c")
    sid = lax.axis_index("s")
    wid = sid * _NC + cid
    pos0 = wid * _P

    def gather_desc(b, ch, buf):
        return pltpu.make_async_copy(
            table_hbm.at[idx_v.at[b, pl.ds(ch * _K, _K)]], row_v.at[buf],
            gsem.at[buf],
        )

    def pe_desc(ch, buf):
        return pltpu.make_async_copy(
            pe_hbm.at[pl.ds(pos0 + ch * _K, _K)], pe_v.at[buf], psem.at[buf]
        )

    def store_desc(b, ch, buf):
        return pltpu.make_async_copy(
            row_v.at[buf], out_hbm.at[b, pl.ds(pos0 + ch * _K, _K)], ssem.at[buf]
        )

    pe_desc(0, 0).start()
    idx_d = [
        pltpu.async_copy(x_hbm.at[b, pl.ds(pos0, _P)], idx_v.at[b], isem)
        for b in range(BATCH)
    ]
    for d in idx_d:
        d.wait()
    gather_desc(0, 0, 0).start()
    gather_desc(1, 0, 1).start()

    @pl.loop(0, _STEPS)
    def _step(t):
        ch = t // BATCH
        b = t % BATCH
        buf = t % 3

        @pl.when(b == 0)
        def _():
            @pl.when(ch + 1 < _NCH)
            def _():
                pe_desc(ch + 1, (ch + 1) % 2).start()

            pe_desc(0, ch % 2).wait()

        gather_desc(b, ch, buf).wait()

        pbuf = ch % 2

        @plsc.parallel_loop(0, _K, unroll=8)
        def _add(i):
            for j in range(_VECS):
                vec = pe_v[pbuf, i, pl.ds(j * _L, _L)]
                plsc.addupdate(row_v.at[buf, i, pl.ds(j * _L, _L)], vec)

        store_desc(b, ch, buf).start()

        @pl.when(t < _STEPS - 2)
        def _():
            nt = t + 2
            nbuf = nt % 3

            @pl.when(t >= 1)
            def _():
                store_desc(0, 0, nbuf).wait()

            gather_desc(nt % BATCH, nt // BATCH, nbuf).start()

    store_desc(0, 0, 0).wait()
    store_desc(0, 0, 1).wait()
    store_desc(0, 0, 2).wait()


@functools.partial(
    pl.kernel,
    out_type=jax.ShapeDtypeStruct((BATCH, SEQ_LEN, D_MODEL), jnp.float32),
    mesh=plsc.VectorSubcoreMesh(
        core_axis_name="c", subcore_axis_name="s", num_cores=_NC, num_subcores=_NS
    ),
    scratch_types=[
        pltpu.VMEM((BATCH, _P), jnp.int32),
        pltpu.VMEM((2, _K, D_MODEL), jnp.float32),
        pltpu.VMEM((3, _K, D_MODEL), jnp.float32),
        pltpu.SemaphoreType.DMA((3,)),
        pltpu.SemaphoreType.DMA((3,)),
        pltpu.SemaphoreType.DMA((2,)),
        pltpu.SemaphoreType.DMA,
    ],
)
def _embed_sc(x_hbm, pe_hbm, table_hbm, out_hbm, idx_v, pe_v, row_v, gsem, ssem, psem, isem):
    _body(x_hbm, pe_hbm, table_hbm, out_hbm, idx_v, pe_v, row_v, gsem, ssem, psem, isem)


@jax.jit
def kernel(x, table):
    return _embed_sc(x, _PE, table)
